# baseline (speedup 1.0000x reference)
import functools

import jax
import jax.numpy as jnp
from jax import lax
from jax.experimental import pallas as pl
from jax.experimental.pallas import tpu as pltpu
from jax.experimental.pallas import tpu_sc as plsc

B = 4096
D = 32
NREL = 64
DD = D * D
NW = 32
BPW = B // NW
RBLK = 512
NBLK = B // RBLK


SUP = 128 // D
OROW = BPW * D // 128


def _extract(sup, off_ref, out_ref):
    iota = lax.broadcasted_iota(jnp.int32, (16,), 0)

    def body(g, carry):
        rows = iota + g * 16
        offs = off_ref[pl.ds(g * 16, 16)]
        nbase = rows * D
        for j in range(D):
            vals = plsc.load_gather(sup, [rows, offs + j])
            n = nbase + j
            plsc.store_scatter(out_ref,
                               [lax.shift_right_logical(n, 7), n & 127], vals)
        return carry

    lax.fori_loop(0, BPW // 16, body, 0)


def _sc_gather_body(users_h, pos_h, neg_h, heads_h, pt_h, nt_h,
                    ue_h, ie_h, ke_h,
                    o_u, o_pi, o_pik, o_ni, o_nik, o_h, o_pt, o_nt,
                    qu, qp, qn, qh, qpt, qnt,
                    fu, fp, fn, fh, fpt, fnt,
                    s0, s1, s2, s3,
                    e0, e1, e2, e3, sem):
    c = lax.axis_index("c")
    s = lax.axis_index("s")
    wid = s * 2 + c
    base = wid * BPW
    obase = wid * OROW
    for hb, qv, fv in ((users_h, qu, fu), (pos_h, qp, fp),
                       (neg_h, qn, fn), (heads_h, qh, fh),
                       (pt_h, qpt, fpt), (nt_h, qnt, fnt)):
        pltpu.sync_copy(hb.at[pl.ds(base, BPW)], qv)
        for t in range(BPW // 16):
            sl = pl.ds(t * 16, 16)
            v = qv[sl]
            fv[sl] = lax.shift_left(v & 3, 5)
            qv[sl] = lax.shift_right_logical(v, 2)
    wave_a = ((ue_h, qu, s0, fu, e0, o_u), (ie_h, qp, s1, fp, e1, o_pi),
              (ke_h, qp, s2, fp, e2, o_pik), (ie_h, qn, s3, fn, e3, o_ni))
    wave_b = ((ke_h, qn, s0, fn, e0, o_nik), (ke_h, qh, s1, fh, e1, o_h),
              (ke_h, qpt, s2, fpt, e2, o_pt), (ke_h, qnt, s3, fnt, e3, o_nt))
    for wave in (wave_a, wave_b):
        cps = [pltpu.async_copy(tbl.at[qv], sv, sem)
               for tbl, qv, sv, _, _, _ in wave]
        for cp in cps:
            cp.wait()
        for tbl, qv, sv, fv, ev, oh in wave:
            _extract(sv, fv, ev)
            pltpu.sync_copy(ev, oh.at[pl.ds(obase, OROW)])


def _sc_gather(users, pos_items, neg_items, heads, pos_tails, neg_tails,
               user_embed, item_embed, kg_entity_embed):
    mesh = plsc.VectorSubcoreMesh(core_axis_name="c", subcore_axis_name="s")
    f = pl.kernel(
        _sc_gather_body,
        out_type=[jax.ShapeDtypeStruct((B * D // 128, 128), jnp.float32)] * 8,
        mesh=mesh,
        scratch_types=(
            [pltpu.VMEM((BPW,), jnp.int32)] * 12
            + [pltpu.VMEM((BPW, 128), jnp.float32)] * 4
            + [pltpu.VMEM((OROW, 128), jnp.float32)] * 4
            + [pltpu.SemaphoreType.DMA]
        ),
        compiler_params=pltpu.CompilerParams(use_tc_tiling_on_sc=True,
                                             needs_layout_passes=False),
    )
    outs = f(users, pos_items, neg_items, heads, pos_tails, neg_tails,
             user_embed.reshape(-1, 128), item_embed.reshape(-1, 128),
             kg_entity_embed.reshape(-1, 128))
    return tuple(o.reshape(B, D) for o in outs)


def _l2n(x):
    n = jnp.sqrt(jnp.sum(x * x, axis=1, keepdims=True))
    return x / jnp.maximum(n, 1e-12)


def _rowwork_body(rel_ref, pie_ref, pik_ref, nie_ref, nik_ref,
                  h_ref, pt_ref, nt_ref, rel_emb_ref, wflat_ref,
                  picomb_ref, nicomb_ref, he_ref, re_ref, pte_ref, nte_ref):
    rel = rel_ref[0, 0, :]
    onehot = (rel[:, None] == lax.broadcasted_iota(jnp.int32, (RBLK, NREL), 1)
              ).astype(jnp.float32)
    re_ref[...] = _l2n(jnp.dot(onehot, rel_emb_ref[...],
                               preferred_element_type=jnp.float32))
    wg = jnp.dot(onehot, wflat_ref[...], preferred_element_type=jnp.float32)
    R = (lax.broadcasted_iota(jnp.int32, (D, DD), 1) % D
         == lax.broadcasted_iota(jnp.int32, (D, DD), 0)).astype(jnp.float32)
    S = (lax.broadcasted_iota(jnp.int32, (DD, D), 0) // D
         == lax.broadcasted_iota(jnp.int32, (DD, D), 1)).astype(jnp.float32)
    for x_ref, o_ref in ((h_ref, he_ref), (pt_ref, pte_ref), (nt_ref, nte_ref)):
        xt = jnp.dot(x_ref[...], R, preferred_element_type=jnp.float32)
        proj = jnp.dot(xt * wg, S, preferred_element_type=jnp.float32)
        o_ref[...] = _l2n(proj)
    picomb_ref[...] = pie_ref[...] + pik_ref[...]
    nicomb_ref[...] = nie_ref[...] + nik_ref[...]


def _matmul_body(u_ref, c_ref, o_ref):
    o_ref[...] = lax.dot_general(u_ref[...], c_ref[...],
                                 (((1,), (1,)), ((), ())),
                                 preferred_element_type=jnp.float32)


def kernel(users, pos_items, neg_items, heads, relations, pos_tails, neg_tails,
           user_embed, item_embed, kg_entity_embed, kg_relation_embed, trans_W):
    u_e, pie, pik, nie, nik, h_raw, pt_raw, nt_raw = _sc_gather(
        users, pos_items, neg_items, heads, pos_tails, neg_tails,
        user_embed, item_embed, kg_entity_embed)
    wflat = trans_W.transpose(0, 2, 1).reshape(NREL, DD)
    rel3 = relations.reshape(NBLK, 1, RBLK)
    row_spec = pl.BlockSpec((RBLK, D), lambda i: (i, 0))
    picomb, nicomb, h_e, r_e, pt_e, nt_e = pl.pallas_call(
        _rowwork_body,
        grid=(NBLK,),
        in_specs=[pl.BlockSpec((1, 1, RBLK), lambda i: (i, 0, 0))]
        + [row_spec] * 7
        + [pl.BlockSpec((NREL, D), lambda i: (0, 0)),
           pl.BlockSpec((NREL, DD), lambda i: (0, 0))],
        out_specs=[row_spec] * 6,
        out_shape=[jax.ShapeDtypeStruct((B, D), jnp.float32)] * 6,
    )(rel3, pie, pik, nie, nik, h_raw, pt_raw, nt_raw,
      kg_relation_embed, wflat)
    preds = pl.pallas_call(
        _matmul_body,
        grid=(NBLK,),
        in_specs=[pl.BlockSpec((RBLK, D), lambda i: (i, 0)),
                  pl.BlockSpec((B, D), lambda i: (0, 0))],
        out_specs=pl.BlockSpec((RBLK, B), lambda i: (i, 0)),
        out_shape=jax.ShapeDtypeStruct((B, B), jnp.float32),
    )(u_e, picomb)
    return (u_e, picomb, nicomb, h_e, r_e, pt_e, nt_e, preds)


# speedup vs baseline: 1.0150x; 1.0150x over previous
import functools

import jax
import jax.numpy as jnp
from jax import lax
from jax.experimental import pallas as pl
from jax.experimental.pallas import tpu as pltpu
from jax.experimental.pallas import tpu_sc as plsc

B = 4096
D = 32
DP = 128
NREL = 64
DD = D * D
NW = 32
BPW = B // NW
RBLK = 512
NBLK = B // RBLK


def _sc_gather_body(users_h, pos_h, neg_h, heads_h, pt_h, nt_h,
                    ue_h, ie_h, ke_h,
                    o_u, o_pi, o_pik, o_ni, o_nik, o_h, o_pt, o_nt,
                    iu, ip, ineg, ih, ipt, int_,
                    r0, r1, r2, r3, sem):
    c = lax.axis_index("c")
    s = lax.axis_index("s")
    wid = s * 2 + c
    base = wid * BPW
    for hb, vb in ((users_h, iu), (pos_h, ip), (neg_h, ineg),
                   (heads_h, ih), (pt_h, ipt), (nt_h, int_)):
        pltpu.sync_copy(hb.at[pl.ds(base, BPW)], vb)
    wave_a = ((ue_h, iu, r0, o_u), (ie_h, ip, r1, o_pi),
              (ke_h, ip, r2, o_pik), (ie_h, ineg, r3, o_ni))
    wave_b = ((ke_h, ineg, r0, o_nik), (ke_h, ih, r1, o_h),
              (ke_h, ipt, r2, o_pt), (ke_h, int_, r3, o_nt))
    for wave in (wave_a, wave_b):
        copies = [pltpu.async_copy(tbl.at[vb], rv, sem)
                  for tbl, vb, rv, _ in wave]
        for cp in copies:
            cp.wait()
        for _, _, rv, oh in wave:
            pltpu.sync_copy(rv, oh.at[pl.ds(base, BPW)])


def _sc_gather(users, pos_items, neg_items, heads, pos_tails, neg_tails,
               user_pad, item_pad, kg_pad):
    mesh = plsc.VectorSubcoreMesh(core_axis_name="c", subcore_axis_name="s")
    f = pl.kernel(
        _sc_gather_body,
        out_type=[jax.ShapeDtypeStruct((B, DP), jnp.float32)] * 8,
        mesh=mesh,
        scratch_types=(
            [pltpu.VMEM((BPW,), jnp.int32)] * 6
            + [pltpu.VMEM((BPW, DP), jnp.float32)] * 4
            + [pltpu.SemaphoreType.DMA]
        ),
        compiler_params=pltpu.CompilerParams(use_tc_tiling_on_sc=True,
                                             needs_layout_passes=False),
    )
    return f(users, pos_items, neg_items, heads, pos_tails, neg_tails,
             user_pad, item_pad, kg_pad)


def _l2n(x):
    n = jnp.sqrt(jnp.sum(x * x, axis=1, keepdims=True))
    return x / jnp.maximum(n, 1e-12)


def _rowwork_body(rel_ref, u_ref, pie_ref, pik_ref, nie_ref, nik_ref,
                  h_ref, pt_ref, nt_ref, rel_emb_ref, wflat_ref,
                  ue_ref, picomb_ref, nicomb_ref, he_ref, re_ref, pte_ref,
                  nte_ref):
    rel = rel_ref[0, 0, :]
    onehot = (rel[:, None] == lax.broadcasted_iota(jnp.int32, (RBLK, NREL), 1)
              ).astype(jnp.float32)
    re_ref[...] = _l2n(jnp.dot(onehot, rel_emb_ref[...],
                               preferred_element_type=jnp.float32))
    wg = jnp.dot(onehot, wflat_ref[...], preferred_element_type=jnp.float32)
    R = (lax.broadcasted_iota(jnp.int32, (D, DD), 1) % D
         == lax.broadcasted_iota(jnp.int32, (D, DD), 0)).astype(jnp.float32)
    S = (lax.broadcasted_iota(jnp.int32, (DD, D), 0) // D
         == lax.broadcasted_iota(jnp.int32, (DD, D), 1)).astype(jnp.float32)
    for x_ref, o_ref in ((h_ref, he_ref), (pt_ref, pte_ref), (nt_ref, nte_ref)):
        xt = jnp.dot(x_ref[:, :D], R, preferred_element_type=jnp.float32)
        proj = jnp.dot(xt * wg, S, preferred_element_type=jnp.float32)
        o_ref[...] = _l2n(proj)
    ue_ref[...] = u_ref[:, :D]
    picomb_ref[...] = pie_ref[:, :D] + pik_ref[:, :D]
    nicomb_ref[...] = nie_ref[:, :D] + nik_ref[:, :D]


def _matmul_body(u_ref, c_ref, o_ref):
    o_ref[...] = lax.dot_general(u_ref[:, :D], c_ref[...],
                                 (((1,), (1,)), ((), ())),
                                 preferred_element_type=jnp.float32)


def kernel(users, pos_items, neg_items, heads, relations, pos_tails, neg_tails,
           user_embed, item_embed, kg_entity_embed, kg_relation_embed, trans_W):
    pad = ((0, 0), (0, DP - D))
    u_pad, i_pad, k_pad = (jnp.pad(user_embed, pad), jnp.pad(item_embed, pad),
                           jnp.pad(kg_entity_embed, pad))
    g_u, g_pi, g_pik, g_ni, g_nik, g_h, g_pt, g_nt = _sc_gather(
        users, pos_items, neg_items, heads, pos_tails, neg_tails,
        u_pad, i_pad, k_pad)
    wflat = trans_W.transpose(0, 2, 1).reshape(NREL, DD)
    rel3 = relations.reshape(NBLK, 1, RBLK)
    row_spec = pl.BlockSpec((RBLK, DP), lambda i: (i, 0))
    out_spec = pl.BlockSpec((RBLK, D), lambda i: (i, 0))
    u_e, picomb, nicomb, h_e, r_e, pt_e, nt_e = pl.pallas_call(
        _rowwork_body,
        grid=(NBLK,),
        in_specs=[pl.BlockSpec((1, 1, RBLK), lambda i: (i, 0, 0))]
        + [row_spec] * 8
        + [pl.BlockSpec((NREL, D), lambda i: (0, 0)),
           pl.BlockSpec((NREL, DD), lambda i: (0, 0))],
        out_specs=[out_spec] * 7,
        out_shape=[jax.ShapeDtypeStruct((B, D), jnp.float32)] * 7,
    )(rel3, g_u, g_pi, g_pik, g_ni, g_nik, g_h, g_pt, g_nt,
      kg_relation_embed, wflat)
    preds = pl.pallas_call(
        _matmul_body,
        grid=(NBLK,),
        in_specs=[pl.BlockSpec((RBLK, DP), lambda i: (i, 0)),
                  pl.BlockSpec((B, D), lambda i: (0, 0))],
        out_specs=pl.BlockSpec((RBLK, B), lambda i: (i, 0)),
        out_shape=jax.ShapeDtypeStruct((B, B), jnp.float32),
    )(g_u, picomb)
    return (u_e, picomb, nicomb, h_e, r_e, pt_e, nt_e, preds)
